# Initial kernel scaffold; baseline (speedup 1.0000x reference)
import jax
import jax.numpy as jnp
from jax.experimental import pallas as pl


def kernel(x, edge_index, W_f, b_f, W_g, b_g, W_h, b_h, W_e, b_e):
    raise NotImplementedError("write your pallas kernel here")


# speedup vs baseline: 3.1234x; 3.1234x over previous
import functools

import jax
import jax.numpy as jnp
from jax import lax
from jax.experimental import pallas as pl
from jax.experimental.pallas import tpu as pltpu
from jax.experimental.pallas import tpu_sc as plsc

N = 10000
E = 320000
IN_DIM = 128
LATENT = 256
EMB = 128
HALF = 128

NPAD = 10240
ROWS_PER_TILE = NPAD // 16
EPT = E // 16
K = 80
CHUNKS = EPT // K


def _sc_deg_body(edges, out, idx, ones, acc):
    c = lax.axis_index("c")
    s = lax.axis_index("s")
    zero16 = jnp.zeros((16,), jnp.float32)

    def zrow(t, _):
        ones[t // 8, pl.ds((t % 8) * 16, 16)] = zero16
        return 0
    lax.fori_loop(0, K * 8, zrow, 0)

    def zacc(i, _):
        pltpu.sync_copy(ones, acc.at[pl.ds(s * ROWS_PER_TILE + i * K, K)])
        return 0
    lax.fori_loop(0, ROWS_PER_TILE // K, zacc, 0)
    plsc.subcore_barrier()

    onehot = jnp.where(lax.iota(jnp.int32, 16) == 0, 1.0, 0.0).astype(jnp.float32)

    def srow(i, _):
        ones[i, pl.ds(0, 16)] = onehot
        return 0
    lax.fori_loop(0, K, srow, 0)

    base0 = c * E + s * EPT

    def chunk(j, _):
        pltpu.sync_copy(edges.at[pl.ds(base0 + j * K, K)], idx)
        pltpu.sync_copy(ones, acc.at[idx], add=True)
        return 0
    lax.fori_loop(0, CHUNKS, chunk, 0)
    plsc.subcore_barrier()

    @pl.when(s < 15)
    def _():
        pltpu.sync_copy(acc.at[pl.ds(s * ROWS_PER_TILE, ROWS_PER_TILE)],
                        out.at[pl.ds(c * N + s * ROWS_PER_TILE, ROWS_PER_TILE)])

    @pl.when(s == 15)
    def _():
        pltpu.sync_copy(acc.at[pl.ds(15 * ROWS_PER_TILE, N - 15 * ROWS_PER_TILE)],
                        out.at[pl.ds(c * N + 15 * ROWS_PER_TILE, N - 15 * ROWS_PER_TILE)])


_sc_deg = functools.partial(
    pl.kernel,
    mesh=plsc.VectorSubcoreMesh(core_axis_name="c", subcore_axis_name="s"),
    out_type=jax.ShapeDtypeStruct((2 * N, HALF), jnp.float32),
    scratch_types=[
        pltpu.VMEM((K,), jnp.int32),
        pltpu.VMEM((K, HALF), jnp.float32),
        pltpu.VMEM_SHARED((NPAD, HALF), jnp.float32),
    ],
)(_sc_deg_body)


def _sc_agg_body(edges, hs, out, sidx, didx, rows, acc, sem):
    c = lax.axis_index("c")
    s = lax.axis_index("s")
    zero16 = jnp.zeros((16,), jnp.float32)

    def zrow(t, _):
        rows[t // 8, pl.ds((t % 8) * 16, 16)] = zero16
        return 0
    lax.fori_loop(0, K * 8, zrow, 0)

    def zacc(i, _):
        pltpu.sync_copy(rows, acc.at[pl.ds(s * ROWS_PER_TILE + i * K, K)])
        return 0
    lax.fori_loop(0, ROWS_PER_TILE // K, zacc, 0)
    plsc.subcore_barrier()

    base0 = s * EPT
    coff = c * N

    def chunk(j, _):
        b = base0 + j * K
        pltpu.sync_copy(edges.at[pl.ds(b, K)], sidx)

        def addoff(t, _):
            sidx[pl.ds(t * 16, 16)] = sidx[pl.ds(t * 16, 16)] + coff
            return 0
        lax.fori_loop(0, K // 16, addoff, 0)
        pltpu.async_copy(hs.at[sidx], rows, sem).wait()
        pltpu.sync_copy(edges.at[pl.ds(E + b, K)], didx)
        pltpu.sync_copy(rows, acc.at[didx], add=True)
        return 0
    lax.fori_loop(0, CHUNKS, chunk, 0)
    plsc.subcore_barrier()

    @pl.when(s < 15)
    def _():
        pltpu.sync_copy(acc.at[pl.ds(s * ROWS_PER_TILE, ROWS_PER_TILE)],
                        out.at[pl.ds(c * N + s * ROWS_PER_TILE, ROWS_PER_TILE)])

    @pl.when(s == 15)
    def _():
        pltpu.sync_copy(acc.at[pl.ds(15 * ROWS_PER_TILE, N - 15 * ROWS_PER_TILE)],
                        out.at[pl.ds(c * N + 15 * ROWS_PER_TILE, N - 15 * ROWS_PER_TILE)])


_sc_agg = functools.partial(
    pl.kernel,
    mesh=plsc.VectorSubcoreMesh(core_axis_name="c", subcore_axis_name="s"),
    out_type=jax.ShapeDtypeStruct((2 * N, HALF), jnp.float32),
    scratch_types=[
        pltpu.VMEM((K,), jnp.int32),
        pltpu.VMEM((K,), jnp.int32),
        pltpu.VMEM((K, HALF), jnp.float32),
        pltpu.VMEM_SHARED((NPAD, HALF), jnp.float32),
        pltpu.SemaphoreType.DMA,
    ],
)(_sc_agg_body)


BN = 1000


def _tc_entry_body(x_ref, wf_ref, bf_ref, dgo_ref, dgi_ref,
                   h_ref, hs_ref, ns_ref, nd_ref):
    ns = lax.rsqrt(jnp.maximum(dgo_ref[...], 1.0))
    nd = lax.rsqrt(jnp.maximum(dgi_ref[...], 1.0))
    h = jnp.dot(x_ref[...], wf_ref[...], preferred_element_type=jnp.float32)
    h = jnp.maximum(h + bf_ref[...], 0.0)
    hs = h * ns
    h_ref[0] = h[:, :HALF]
    h_ref[1] = h[:, HALF:]
    hs_ref[0] = hs[:, :HALF]
    hs_ref[1] = hs[:, HALF:]
    ns_ref[...] = ns
    nd_ref[...] = nd


_tc_entry = pl.pallas_call(
    _tc_entry_body,
    grid=(N // BN,),
    in_specs=[
        pl.BlockSpec((BN, IN_DIM), lambda i: (i, 0)),
        pl.BlockSpec((IN_DIM, LATENT), lambda i: (0, 0)),
        pl.BlockSpec((1, LATENT), lambda i: (0, 0)),
        pl.BlockSpec((BN, 1), lambda i: (i, 0)),
        pl.BlockSpec((BN, 1), lambda i: (i, 0)),
    ],
    out_specs=[
        pl.BlockSpec((2, BN, HALF), lambda i: (0, i, 0)),
        pl.BlockSpec((2, BN, HALF), lambda i: (0, i, 0)),
        pl.BlockSpec((BN, 1), lambda i: (i, 0)),
        pl.BlockSpec((BN, 1), lambda i: (i, 0)),
    ],
    out_shape=[
        jax.ShapeDtypeStruct((2, N, HALF), jnp.float32),
        jax.ShapeDtypeStruct((2, N, HALF), jnp.float32),
        jax.ShapeDtypeStruct((N, 1), jnp.float32),
        jax.ShapeDtypeStruct((N, 1), jnp.float32),
    ],
)


def _tc_layer_body(agg_ref, h_ref, nd_ref, ns_ref, w_ref, b_ref,
                   ho_ref, hso_ref):
    agg = jnp.concatenate([agg_ref[0], agg_ref[1]], axis=1) * nd_ref[...]
    y = jnp.dot(agg, w_ref[...], preferred_element_type=jnp.float32)
    y = jnp.maximum(y + b_ref[...], 0.0)
    h = jnp.concatenate([h_ref[0], h_ref[1]], axis=1) + y
    hs = h * ns_ref[...]
    ho_ref[0] = h[:, :HALF]
    ho_ref[1] = h[:, HALF:]
    hso_ref[0] = hs[:, :HALF]
    hso_ref[1] = hs[:, HALF:]


_tc_layer = pl.pallas_call(
    _tc_layer_body,
    grid=(N // BN,),
    in_specs=[
        pl.BlockSpec((2, BN, HALF), lambda i: (0, i, 0)),
        pl.BlockSpec((2, BN, HALF), lambda i: (0, i, 0)),
        pl.BlockSpec((BN, 1), lambda i: (i, 0)),
        pl.BlockSpec((BN, 1), lambda i: (i, 0)),
        pl.BlockSpec((LATENT, LATENT), lambda i: (0, 0)),
        pl.BlockSpec((1, LATENT), lambda i: (0, 0)),
    ],
    out_specs=[
        pl.BlockSpec((2, BN, HALF), lambda i: (0, i, 0)),
        pl.BlockSpec((2, BN, HALF), lambda i: (0, i, 0)),
    ],
    out_shape=[
        jax.ShapeDtypeStruct((2, N, HALF), jnp.float32),
        jax.ShapeDtypeStruct((2, N, HALF), jnp.float32),
    ],
)


def _tc_final_body(hrow_ref, we_ref, be_ref, o_ref):
    hr = jnp.concatenate([hrow_ref[0], hrow_ref[1]], axis=1)
    o = jnp.dot(hr, we_ref[...], preferred_element_type=jnp.float32)
    o_ref[...] = jnp.maximum(o + be_ref[...], 0.0)


_tc_final = pl.pallas_call(
    _tc_final_body,
    out_shape=jax.ShapeDtypeStruct((8, EMB), jnp.float32),
)


def kernel(x, edge_index, W_f, b_f, W_g, b_g, W_h, b_h, W_e, b_e):
    edges_flat = edge_index.reshape(2 * E)
    degs = _sc_deg(edges_flat)
    dgo = degs[0:N, 0].reshape(N, 1)
    dgi = degs[N:2 * N, 0].reshape(N, 1)

    h, hs, ns, nd = _tc_entry(x, W_f, b_f.reshape(1, LATENT), dgo, dgi)
    for W, b, steps in ((W_g, b_g, 2), (W_h, b_h, 4)):
        bb = b.reshape(1, LATENT)
        for _ in range(steps):
            agg = _sc_agg(edges_flat, hs.reshape(2 * N, HALF))
            h, hs = _tc_layer(agg.reshape(2, N, HALF), h, nd, ns, W, bb)

    out8 = _tc_final(h[:, 0:8, :], W_e, b_e.reshape(1, EMB))
    return out8[0]


# speedup vs baseline: 5.8725x; 1.8802x over previous
import functools

import jax
import jax.numpy as jnp
from jax import lax
from jax.experimental import pallas as pl
from jax.experimental.pallas import tpu as pltpu
from jax.experimental.pallas import tpu_sc as plsc

N = 10000
E = 320000
IN_DIM = 128
LATENT = 256
EMB = 128
HALF = 128

NPAD = 10240
ROWS_PER_TILE = NPAD // 16
EPT = E // 16
K = 80
CHUNKS = EPT // K
PHASES = 5
CPP = CHUNKS // PHASES
IDS_PP = CPP * K


def _refill(idx, ev, j):
    def go(t, _):
        idx[pl.ds(t * 16, 16)] = ev[pl.ds(j * K + t * 16, 16)]
        return 0
    lax.fori_loop(0, K // 16, go, 0)


def _sc_deg_body(edges, out, ev, idx0, idx1, ones, acc, ssem0, ssem1):
    c = lax.axis_index("c")
    s = lax.axis_index("s")
    zero16 = jnp.zeros((16,), jnp.float32)

    def zrow(t, _):
        ones[t // 8, pl.ds((t % 8) * 16, 16)] = zero16
        return 0
    lax.fori_loop(0, K * 8, zrow, 0)

    def zacc(i, _):
        pltpu.sync_copy(ones, acc.at[pl.ds(s * ROWS_PER_TILE + i * K, K)])
        return 0
    lax.fori_loop(0, ROWS_PER_TILE // K, zacc, 0)
    plsc.subcore_barrier()

    onehot = jnp.where(lax.iota(jnp.int32, 16) == 0, 1.0, 0.0).astype(jnp.float32)

    def srow(i, _):
        ones[i, pl.ds(0, 16)] = onehot
        return 0
    lax.fori_loop(0, K, srow, 0)

    pltpu.sync_copy(edges.at[pl.ds(c * E + s * EPT, EPT)], ev)
    _refill(idx0, ev, 0)

    def body(i, _):
        a = 2 * i
        pltpu.async_copy(ones, acc.at[idx0], ssem0, add=True)

        @pl.when(i > 0)
        def _():
            pltpu.make_async_copy(ones, acc.at[idx1], ssem1).wait()
        _refill(idx1, ev, a + 1)
        pltpu.async_copy(ones, acc.at[idx1], ssem1, add=True)
        pltpu.make_async_copy(ones, acc.at[idx0], ssem0).wait()

        @pl.when(i < CHUNKS // 2 - 1)
        def _():
            _refill(idx0, ev, a + 2)
        return 0
    lax.fori_loop(0, CHUNKS // 2, body, 0)
    pltpu.make_async_copy(ones, acc.at[idx1], ssem1).wait()
    plsc.subcore_barrier()

    @pl.when(s < 15)
    def _():
        pltpu.sync_copy(acc.at[pl.ds(s * ROWS_PER_TILE, ROWS_PER_TILE)],
                        out.at[pl.ds(c * N + s * ROWS_PER_TILE, ROWS_PER_TILE)])

    @pl.when(s == 15)
    def _():
        pltpu.sync_copy(acc.at[pl.ds(15 * ROWS_PER_TILE, N - 15 * ROWS_PER_TILE)],
                        out.at[pl.ds(c * N + 15 * ROWS_PER_TILE, N - 15 * ROWS_PER_TILE)])


_sc_deg = functools.partial(
    pl.kernel,
    mesh=plsc.VectorSubcoreMesh(core_axis_name="c", subcore_axis_name="s"),
    out_type=jax.ShapeDtypeStruct((2 * N, HALF), jnp.float32),
    scratch_types=[
        pltpu.VMEM((EPT,), jnp.int32),
        pltpu.VMEM((K,), jnp.int32),
        pltpu.VMEM((K,), jnp.int32),
        pltpu.VMEM((K, HALF), jnp.float32),
        pltpu.VMEM_SHARED((NPAD, HALF), jnp.float32),
        pltpu.SemaphoreType.DMA,
        pltpu.SemaphoreType.DMA,
    ],
)(_sc_deg_body)


def _sc_agg_body(edges, hs, out, sv, dv, didx0, didx1, rows, acc,
                 gsem0, gsem1, ssem0, ssem1):
    c = lax.axis_index("c")
    s = lax.axis_index("s")
    zero16 = jnp.zeros((16,), jnp.float32)

    def zrow(t, _):
        rows[0, t // 8, pl.ds((t % 8) * 16, 16)] = zero16
        return 0
    lax.fori_loop(0, K * 8, zrow, 0)

    def zacc(i, _):
        pltpu.sync_copy(rows.at[0], acc.at[pl.ds(s * ROWS_PER_TILE + i * K, K)])
        return 0
    lax.fori_loop(0, ROWS_PER_TILE // K, zacc, 0)
    plsc.subcore_barrier()

    eb = s * EPT
    coff = c * N

    def gather(j, slot, sem):
        return pltpu.async_copy(hs.at[sv.at[pl.ds(j * K, K)]], rows.at[slot], sem)

    def gwait(slot, sem):
        pltpu.make_async_copy(hs.at[sv.at[pl.ds(0, K)]], rows.at[slot], sem).wait()

    def swait(slot, idx, sem):
        pltpu.make_async_copy(rows.at[slot], acc.at[idx], sem).wait()

    def phase(p, _):
        base = eb + p * IDS_PP
        pltpu.sync_copy(edges.at[pl.ds(base, IDS_PP)], sv)
        pltpu.sync_copy(edges.at[pl.ds(E + base, IDS_PP)], dv)

        def addoff(t, _):
            sv[pl.ds(t * 16, 16)] = sv[pl.ds(t * 16, 16)] + coff
            return 0
        lax.fori_loop(0, IDS_PP // 16, addoff, 0)

        _refill(didx0, dv, 0)
        gather(0, 0, gsem0)

        def body(i, _):
            a = 2 * i
            gwait(0, gsem0)
            pltpu.async_copy(rows.at[0], acc.at[didx0], ssem0, add=True)

            @pl.when(i > 0)
            def _():
                swait(1, didx1, ssem1)
            gather(a + 1, 1, gsem1)
            _refill(didx1, dv, a + 1)
            gwait(1, gsem1)
            pltpu.async_copy(rows.at[1], acc.at[didx1], ssem1, add=True)
            swait(0, didx0, ssem0)

            @pl.when(i < CPP // 2 - 1)
            def _():
                _refill(didx0, dv, a + 2)
                gather(a + 2, 0, gsem0)
            return 0
        lax.fori_loop(0, CPP // 2, body, 0)
        swait(1, didx1, ssem1)
        return 0
    lax.fori_loop(0, PHASES, phase, 0)
    plsc.subcore_barrier()

    @pl.when(s < 15)
    def _():
        pltpu.sync_copy(acc.at[pl.ds(s * ROWS_PER_TILE, ROWS_PER_TILE)],
                        out.at[pl.ds(c * N + s * ROWS_PER_TILE, ROWS_PER_TILE)])

    @pl.when(s == 15)
    def _():
        pltpu.sync_copy(acc.at[pl.ds(15 * ROWS_PER_TILE, N - 15 * ROWS_PER_TILE)],
                        out.at[pl.ds(c * N + 15 * ROWS_PER_TILE, N - 15 * ROWS_PER_TILE)])


_sc_agg = functools.partial(
    pl.kernel,
    mesh=plsc.VectorSubcoreMesh(core_axis_name="c", subcore_axis_name="s"),
    out_type=jax.ShapeDtypeStruct((2 * N, HALF), jnp.float32),
    scratch_types=[
        pltpu.VMEM((IDS_PP,), jnp.int32),
        pltpu.VMEM((IDS_PP,), jnp.int32),
        pltpu.VMEM((K,), jnp.int32),
        pltpu.VMEM((K,), jnp.int32),
        pltpu.VMEM((2, K, HALF), jnp.float32),
        pltpu.VMEM_SHARED((NPAD, HALF), jnp.float32),
        pltpu.SemaphoreType.DMA,
        pltpu.SemaphoreType.DMA,
        pltpu.SemaphoreType.DMA,
        pltpu.SemaphoreType.DMA,
    ],
)(_sc_agg_body)


BN = 1000


def _tc_entry_body(x_ref, wf_ref, bf_ref, dgo_ref, dgi_ref,
                   h_ref, hs_ref, ns_ref, nd_ref):
    ns = lax.rsqrt(jnp.maximum(dgo_ref[...], 1.0))
    nd = lax.rsqrt(jnp.maximum(dgi_ref[...], 1.0))
    h = jnp.dot(x_ref[...], wf_ref[...], preferred_element_type=jnp.float32)
    h = jnp.maximum(h + bf_ref[...], 0.0)
    hs = h * ns
    h_ref[0] = h[:, :HALF]
    h_ref[1] = h[:, HALF:]
    hs_ref[0] = hs[:, :HALF]
    hs_ref[1] = hs[:, HALF:]
    ns_ref[...] = ns
    nd_ref[...] = nd


_tc_entry = pl.pallas_call(
    _tc_entry_body,
    grid=(N // BN,),
    in_specs=[
        pl.BlockSpec((BN, IN_DIM), lambda i: (i, 0)),
        pl.BlockSpec((IN_DIM, LATENT), lambda i: (0, 0)),
        pl.BlockSpec((1, LATENT), lambda i: (0, 0)),
        pl.BlockSpec((BN, 1), lambda i: (i, 0)),
        pl.BlockSpec((BN, 1), lambda i: (i, 0)),
    ],
    out_specs=[
        pl.BlockSpec((2, BN, HALF), lambda i: (0, i, 0)),
        pl.BlockSpec((2, BN, HALF), lambda i: (0, i, 0)),
        pl.BlockSpec((BN, 1), lambda i: (i, 0)),
        pl.BlockSpec((BN, 1), lambda i: (i, 0)),
    ],
    out_shape=[
        jax.ShapeDtypeStruct((2, N, HALF), jnp.float32),
        jax.ShapeDtypeStruct((2, N, HALF), jnp.float32),
        jax.ShapeDtypeStruct((N, 1), jnp.float32),
        jax.ShapeDtypeStruct((N, 1), jnp.float32),
    ],
)


def _tc_layer_body(agg_ref, h_ref, nd_ref, ns_ref, w_ref, b_ref,
                   ho_ref, hso_ref):
    agg = jnp.concatenate([agg_ref[0], agg_ref[1]], axis=1) * nd_ref[...]
    y = jnp.dot(agg, w_ref[...], preferred_element_type=jnp.float32)
    y = jnp.maximum(y + b_ref[...], 0.0)
    h = jnp.concatenate([h_ref[0], h_ref[1]], axis=1) + y
    hs = h * ns_ref[...]
    ho_ref[0] = h[:, :HALF]
    ho_ref[1] = h[:, HALF:]
    hso_ref[0] = hs[:, :HALF]
    hso_ref[1] = hs[:, HALF:]


_tc_layer = pl.pallas_call(
    _tc_layer_body,
    grid=(N // BN,),
    in_specs=[
        pl.BlockSpec((2, BN, HALF), lambda i: (0, i, 0)),
        pl.BlockSpec((2, BN, HALF), lambda i: (0, i, 0)),
        pl.BlockSpec((BN, 1), lambda i: (i, 0)),
        pl.BlockSpec((BN, 1), lambda i: (i, 0)),
        pl.BlockSpec((LATENT, LATENT), lambda i: (0, 0)),
        pl.BlockSpec((1, LATENT), lambda i: (0, 0)),
    ],
    out_specs=[
        pl.BlockSpec((2, BN, HALF), lambda i: (0, i, 0)),
        pl.BlockSpec((2, BN, HALF), lambda i: (0, i, 0)),
    ],
    out_shape=[
        jax.ShapeDtypeStruct((2, N, HALF), jnp.float32),
        jax.ShapeDtypeStruct((2, N, HALF), jnp.float32),
    ],
)


def _tc_final_body(hrow_ref, we_ref, be_ref, o_ref):
    hr = jnp.concatenate([hrow_ref[0], hrow_ref[1]], axis=1)
    o = jnp.dot(hr, we_ref[...], preferred_element_type=jnp.float32)
    o_ref[...] = jnp.maximum(o + be_ref[...], 0.0)


_tc_final = pl.pallas_call(
    _tc_final_body,
    out_shape=jax.ShapeDtypeStruct((8, EMB), jnp.float32),
)


def kernel(x, edge_index, W_f, b_f, W_g, b_g, W_h, b_h, W_e, b_e):
    edges_flat = edge_index.reshape(2 * E)
    degs = _sc_deg(edges_flat)
    dgo = degs[0:N, 0].reshape(N, 1)
    dgi = degs[N:2 * N, 0].reshape(N, 1)

    h, hs, ns, nd = _tc_entry(x, W_f, b_f.reshape(1, LATENT), dgo, dgi)
    for W, b, steps in ((W_g, b_g, 2), (W_h, b_h, 4)):
        bb = b.reshape(1, LATENT)
        for _ in range(steps):
            agg = _sc_agg(edges_flat, hs.reshape(2 * N, HALF))
            h, hs = _tc_layer(agg.reshape(2, N, HALF), h, nd, ns, W, bb)

    out8 = _tc_final(h[:, 0:8, :], W_e, b_e.reshape(1, EMB))
    return out8[0]


# speedup vs baseline: 5.9748x; 1.0174x over previous
import functools

import jax
import jax.numpy as jnp
from jax import lax
from jax.experimental import pallas as pl
from jax.experimental.pallas import tpu as pltpu
from jax.experimental.pallas import tpu_sc as plsc

N = 10000
E = 320000
IN_DIM = 128
LATENT = 256
EMB = 128
HALF = 128

NPAD = 10240
ROWS_PER_TILE = NPAD // 16
EPT = E // 16
K = 80
CHUNKS = EPT // K
PHASES = 5
CPP = CHUNKS // PHASES
IDS_PP = CPP * K


def _refill(idx, ev, j):
    def go(t, _):
        idx[pl.ds(t * 16, 16)] = ev[pl.ds(j * K + t * 16, 16)]
        return 0
    lax.fori_loop(0, K // 16, go, 0)


def _sc_deg_body(edges, out, ev, idx0, idx1, ones, acc, ssem0, ssem1):
    c = lax.axis_index("c")
    s = lax.axis_index("s")
    zero16 = jnp.zeros((16,), jnp.float32)

    def zrow(t, _):
        ones[t // 8, pl.ds((t % 8) * 16, 16)] = zero16
        return 0
    lax.fori_loop(0, K * 8, zrow, 0)

    def zacc(i, _):
        pltpu.sync_copy(ones, acc.at[pl.ds(s * ROWS_PER_TILE + i * K, K)])
        return 0
    lax.fori_loop(0, ROWS_PER_TILE // K, zacc, 0)
    plsc.subcore_barrier()

    onehot = jnp.where(lax.iota(jnp.int32, 16) == 0, 1.0, 0.0).astype(jnp.float32)

    def srow(i, _):
        ones[i, pl.ds(0, 16)] = onehot
        return 0
    lax.fori_loop(0, K, srow, 0)

    pltpu.sync_copy(edges.at[pl.ds(c * E + s * EPT, EPT)], ev)
    _refill(idx0, ev, 0)

    def body(i, _):
        a = 2 * i
        pltpu.async_copy(ones, acc.at[idx0], ssem0, add=True)

        @pl.when(i > 0)
        def _():
            pltpu.make_async_copy(ones, acc.at[idx1], ssem1).wait()
        _refill(idx1, ev, a + 1)
        pltpu.async_copy(ones, acc.at[idx1], ssem1, add=True)
        pltpu.make_async_copy(ones, acc.at[idx0], ssem0).wait()

        @pl.when(i < CHUNKS // 2 - 1)
        def _():
            _refill(idx0, ev, a + 2)
        return 0
    lax.fori_loop(0, CHUNKS // 2, body, 0)
    pltpu.make_async_copy(ones, acc.at[idx1], ssem1).wait()
    plsc.subcore_barrier()

    @pl.when(s < 15)
    def _():
        pltpu.sync_copy(acc.at[pl.ds(s * ROWS_PER_TILE, ROWS_PER_TILE)],
                        out.at[pl.ds(c * N + s * ROWS_PER_TILE, ROWS_PER_TILE)])

    @pl.when(s == 15)
    def _():
        pltpu.sync_copy(acc.at[pl.ds(15 * ROWS_PER_TILE, N - 15 * ROWS_PER_TILE)],
                        out.at[pl.ds(c * N + 15 * ROWS_PER_TILE, N - 15 * ROWS_PER_TILE)])


_sc_deg = functools.partial(
    pl.kernel,
    mesh=plsc.VectorSubcoreMesh(core_axis_name="c", subcore_axis_name="s"),
    out_type=jax.ShapeDtypeStruct((2 * N, HALF), jnp.float32),
    scratch_types=[
        pltpu.VMEM((EPT,), jnp.int32),
        pltpu.VMEM((K,), jnp.int32),
        pltpu.VMEM((K,), jnp.int32),
        pltpu.VMEM((K, HALF), jnp.float32),
        pltpu.VMEM_SHARED((NPAD, HALF), jnp.float32),
        pltpu.SemaphoreType.DMA,
        pltpu.SemaphoreType.DMA,
    ],
)(_sc_deg_body)


def _sc_agg_body(edges, hs, out, sv, dv, didx0, didx1, rows, acc,
                 gsem0, gsem1, ssem0, ssem1):
    c = lax.axis_index("c")
    s = lax.axis_index("s")
    zero16 = jnp.zeros((16,), jnp.float32)

    def zrow(t, _):
        rows[0, t // 8, pl.ds((t % 8) * 16, 16)] = zero16
        return 0
    lax.fori_loop(0, K * 8, zrow, 0)

    def zacc(i, _):
        pltpu.sync_copy(rows.at[0], acc.at[pl.ds(s * ROWS_PER_TILE + i * K, K)])
        return 0
    lax.fori_loop(0, ROWS_PER_TILE // K, zacc, 0)
    plsc.subcore_barrier()

    eb = s * EPT
    coff = c * N

    def gather(j, slot, sem):
        return pltpu.async_copy(hs.at[sv.at[pl.ds(j * K, K)]], rows.at[slot], sem)

    def gwait(slot, sem):
        pltpu.make_async_copy(hs.at[sv.at[pl.ds(0, K)]], rows.at[slot], sem).wait()

    def swait(slot, idx, sem):
        pltpu.make_async_copy(rows.at[slot], acc.at[idx], sem).wait()

    def phase(p, _):
        base = eb + p * IDS_PP
        pltpu.sync_copy(edges.at[pl.ds(base, IDS_PP)], sv)
        pltpu.sync_copy(edges.at[pl.ds(E + base, IDS_PP)], dv)

        def addoff(t, _):
            sv[pl.ds(t * 16, 16)] = sv[pl.ds(t * 16, 16)] + coff
            return 0
        lax.fori_loop(0, IDS_PP // 16, addoff, 0)

        _refill(didx0, dv, 0)
        _refill(didx1, dv, 1)
        gather(0, 0, gsem0)
        gather(1, 1, gsem1)

        def body(i, _):
            a = 2 * i
            gwait(0, gsem0)
            pltpu.async_copy(rows.at[0], acc.at[didx0], ssem0, add=True)
            gwait(1, gsem1)
            pltpu.async_copy(rows.at[1], acc.at[didx1], ssem1, add=True)
            swait(0, didx0, ssem0)

            @pl.when(i < CPP // 2 - 1)
            def _():
                _refill(didx0, dv, a + 2)
                gather(a + 2, 0, gsem0)
            swait(1, didx1, ssem1)

            @pl.when(i < CPP // 2 - 1)
            def _():
                _refill(didx1, dv, a + 3)
                gather(a + 3, 1, gsem1)
            return 0
        lax.fori_loop(0, CPP // 2, body, 0)
        return 0
    lax.fori_loop(0, PHASES, phase, 0)
    plsc.subcore_barrier()

    @pl.when(s < 15)
    def _():
        pltpu.sync_copy(acc.at[pl.ds(s * ROWS_PER_TILE, ROWS_PER_TILE)],
                        out.at[pl.ds(c * N + s * ROWS_PER_TILE, ROWS_PER_TILE)])

    @pl.when(s == 15)
    def _():
        pltpu.sync_copy(acc.at[pl.ds(15 * ROWS_PER_TILE, N - 15 * ROWS_PER_TILE)],
                        out.at[pl.ds(c * N + 15 * ROWS_PER_TILE, N - 15 * ROWS_PER_TILE)])


_sc_agg = functools.partial(
    pl.kernel,
    mesh=plsc.VectorSubcoreMesh(core_axis_name="c", subcore_axis_name="s"),
    out_type=jax.ShapeDtypeStruct((2 * N, HALF), jnp.float32),
    scratch_types=[
        pltpu.VMEM((IDS_PP,), jnp.int32),
        pltpu.VMEM((IDS_PP,), jnp.int32),
        pltpu.VMEM((K,), jnp.int32),
        pltpu.VMEM((K,), jnp.int32),
        pltpu.VMEM((2, K, HALF), jnp.float32),
        pltpu.VMEM_SHARED((NPAD, HALF), jnp.float32),
        pltpu.SemaphoreType.DMA,
        pltpu.SemaphoreType.DMA,
        pltpu.SemaphoreType.DMA,
        pltpu.SemaphoreType.DMA,
    ],
)(_sc_agg_body)


BN = 1000


def _tc_entry_body(x_ref, wf_ref, bf_ref, dgo_ref, dgi_ref,
                   h_ref, hs_ref, ns_ref, nd_ref):
    ns = lax.rsqrt(jnp.maximum(dgo_ref[...], 1.0))
    nd = lax.rsqrt(jnp.maximum(dgi_ref[...], 1.0))
    h = jnp.dot(x_ref[...], wf_ref[...], preferred_element_type=jnp.float32)
    h = jnp.maximum(h + bf_ref[...], 0.0)
    hs = h * ns
    h_ref[0] = h[:, :HALF]
    h_ref[1] = h[:, HALF:]
    hs_ref[0] = hs[:, :HALF]
    hs_ref[1] = hs[:, HALF:]
    ns_ref[...] = ns
    nd_ref[...] = nd


_tc_entry = pl.pallas_call(
    _tc_entry_body,
    grid=(N // BN,),
    in_specs=[
        pl.BlockSpec((BN, IN_DIM), lambda i: (i, 0)),
        pl.BlockSpec((IN_DIM, LATENT), lambda i: (0, 0)),
        pl.BlockSpec((1, LATENT), lambda i: (0, 0)),
        pl.BlockSpec((BN, 1), lambda i: (i, 0)),
        pl.BlockSpec((BN, 1), lambda i: (i, 0)),
    ],
    out_specs=[
        pl.BlockSpec((2, BN, HALF), lambda i: (0, i, 0)),
        pl.BlockSpec((2, BN, HALF), lambda i: (0, i, 0)),
        pl.BlockSpec((BN, 1), lambda i: (i, 0)),
        pl.BlockSpec((BN, 1), lambda i: (i, 0)),
    ],
    out_shape=[
        jax.ShapeDtypeStruct((2, N, HALF), jnp.float32),
        jax.ShapeDtypeStruct((2, N, HALF), jnp.float32),
        jax.ShapeDtypeStruct((N, 1), jnp.float32),
        jax.ShapeDtypeStruct((N, 1), jnp.float32),
    ],
)


def _tc_layer_body(agg_ref, h_ref, nd_ref, ns_ref, w_ref, b_ref,
                   ho_ref, hso_ref):
    agg = jnp.concatenate([agg_ref[0], agg_ref[1]], axis=1) * nd_ref[...]
    y = jnp.dot(agg, w_ref[...], preferred_element_type=jnp.float32)
    y = jnp.maximum(y + b_ref[...], 0.0)
    h = jnp.concatenate([h_ref[0], h_ref[1]], axis=1) + y
    hs = h * ns_ref[...]
    ho_ref[0] = h[:, :HALF]
    ho_ref[1] = h[:, HALF:]
    hso_ref[0] = hs[:, :HALF]
    hso_ref[1] = hs[:, HALF:]


_tc_layer = pl.pallas_call(
    _tc_layer_body,
    grid=(N // BN,),
    in_specs=[
        pl.BlockSpec((2, BN, HALF), lambda i: (0, i, 0)),
        pl.BlockSpec((2, BN, HALF), lambda i: (0, i, 0)),
        pl.BlockSpec((BN, 1), lambda i: (i, 0)),
        pl.BlockSpec((BN, 1), lambda i: (i, 0)),
        pl.BlockSpec((LATENT, LATENT), lambda i: (0, 0)),
        pl.BlockSpec((1, LATENT), lambda i: (0, 0)),
    ],
    out_specs=[
        pl.BlockSpec((2, BN, HALF), lambda i: (0, i, 0)),
        pl.BlockSpec((2, BN, HALF), lambda i: (0, i, 0)),
    ],
    out_shape=[
        jax.ShapeDtypeStruct((2, N, HALF), jnp.float32),
        jax.ShapeDtypeStruct((2, N, HALF), jnp.float32),
    ],
)


def _tc_final_body(hrow_ref, we_ref, be_ref, o_ref):
    hr = jnp.concatenate([hrow_ref[0], hrow_ref[1]], axis=1)
    o = jnp.dot(hr, we_ref[...], preferred_element_type=jnp.float32)
    o_ref[...] = jnp.maximum(o + be_ref[...], 0.0)


_tc_final = pl.pallas_call(
    _tc_final_body,
    out_shape=jax.ShapeDtypeStruct((8, EMB), jnp.float32),
)


def kernel(x, edge_index, W_f, b_f, W_g, b_g, W_h, b_h, W_e, b_e):
    edges_flat = edge_index.reshape(2 * E)
    degs = _sc_deg(edges_flat)
    dgo = degs[0:N, 0].reshape(N, 1)
    dgi = degs[N:2 * N, 0].reshape(N, 1)

    h, hs, ns, nd = _tc_entry(x, W_f, b_f.reshape(1, LATENT), dgo, dgi)
    for W, b, steps in ((W_g, b_g, 2), (W_h, b_h, 4)):
        bb = b.reshape(1, LATENT)
        for _ in range(steps):
            agg = _sc_agg(edges_flat, hs.reshape(2 * N, HALF))
            h, hs = _tc_layer(agg.reshape(2, N, HALF), h, nd, ns, W, bb)

    out8 = _tc_final(h[:, 0:8, :], W_e, b_e.reshape(1, EMB))
    return out8[0]


# speedup vs baseline: 6.6476x; 1.1126x over previous
import functools

import jax
import jax.numpy as jnp
from jax import lax
from jax.experimental import pallas as pl
from jax.experimental.pallas import tpu as pltpu
from jax.experimental.pallas import tpu_sc as plsc

N = 10000
E = 320000
IN_DIM = 128
LATENT = 256
EMB = 128
HALF = 128

NPAD = 10240
ROWS_PER_TILE = NPAD // 16
EPT = E // 16
K = 80
CHUNKS = EPT // K
PHASES = 5
CPP = CHUNKS // PHASES
IDS_PP = CPP * K


def _refill(idx, ev, j):
    def go(t, _):
        idx[pl.ds(t * 16, 16)] = ev[pl.ds(j * K + t * 16, 16)]
        return 0
    lax.fori_loop(0, K // 16, go, 0)


def _sc_deg_body(edges, out, ev, idx0, idx1, ones, acc, ssem0, ssem1):
    c = lax.axis_index("c")
    s = lax.axis_index("s")
    zero16 = jnp.zeros((16,), jnp.float32)

    def zrow(t, _):
        ones[t // 8, pl.ds((t % 8) * 16, 16)] = zero16
        return 0
    lax.fori_loop(0, K * 8, zrow, 0)

    def zacc(i, _):
        pltpu.sync_copy(ones, acc.at[pl.ds(s * ROWS_PER_TILE + i * K, K)])
        return 0
    lax.fori_loop(0, ROWS_PER_TILE // K, zacc, 0)
    plsc.subcore_barrier()

    onehot = jnp.where(lax.iota(jnp.int32, 16) == 0, 1.0, 0.0).astype(jnp.float32)

    def srow(i, _):
        ones[i, pl.ds(0, 16)] = onehot
        return 0
    lax.fori_loop(0, K, srow, 0)

    pltpu.sync_copy(edges.at[pl.ds(c * E + s * EPT, EPT)], ev)
    _refill(idx0, ev, 0)

    def body(i, _):
        a = 2 * i
        pltpu.async_copy(ones, acc.at[idx0], ssem0, add=True)

        @pl.when(i > 0)
        def _():
            pltpu.make_async_copy(ones, acc.at[idx1], ssem1).wait()
        _refill(idx1, ev, a + 1)
        pltpu.async_copy(ones, acc.at[idx1], ssem1, add=True)
        pltpu.make_async_copy(ones, acc.at[idx0], ssem0).wait()

        @pl.when(i < CHUNKS // 2 - 1)
        def _():
            _refill(idx0, ev, a + 2)
        return 0
    lax.fori_loop(0, CHUNKS // 2, body, 0)
    pltpu.make_async_copy(ones, acc.at[idx1], ssem1).wait()
    plsc.subcore_barrier()

    @pl.when(s < 15)
    def _():
        pltpu.sync_copy(acc.at[pl.ds(s * ROWS_PER_TILE, ROWS_PER_TILE)],
                        out.at[pl.ds(c * N + s * ROWS_PER_TILE, ROWS_PER_TILE)])

    @pl.when(s == 15)
    def _():
        pltpu.sync_copy(acc.at[pl.ds(15 * ROWS_PER_TILE, N - 15 * ROWS_PER_TILE)],
                        out.at[pl.ds(c * N + 15 * ROWS_PER_TILE, N - 15 * ROWS_PER_TILE)])


_sc_deg = functools.partial(
    pl.kernel,
    mesh=plsc.VectorSubcoreMesh(core_axis_name="c", subcore_axis_name="s"),
    out_type=jax.ShapeDtypeStruct((2 * N, HALF), jnp.float32),
    scratch_types=[
        pltpu.VMEM((EPT,), jnp.int32),
        pltpu.VMEM((K,), jnp.int32),
        pltpu.VMEM((K,), jnp.int32),
        pltpu.VMEM((K, HALF), jnp.float32),
        pltpu.VMEM_SHARED((NPAD, HALF), jnp.float32),
        pltpu.SemaphoreType.DMA,
        pltpu.SemaphoreType.DMA,
    ],
)(_sc_deg_body)


def _sc_agg_body(edges, hs, out, sv, dv, didx0, didx1, rows, acc,
                 gsem0, gsem1, ssem0, ssem1):
    c = lax.axis_index("c")
    s = lax.axis_index("s")
    zero16 = jnp.zeros((16,), jnp.float32)

    def zrow(t, _):
        rows[0, t // 8, pl.ds((t % 8) * 16, 16)] = zero16
        return 0
    lax.fori_loop(0, K * 8, zrow, 0)

    def zacc(i, _):
        pltpu.sync_copy(rows.at[0], acc.at[pl.ds(s * ROWS_PER_TILE + i * K, K)])
        return 0
    lax.fori_loop(0, ROWS_PER_TILE // K, zacc, 0)
    plsc.subcore_barrier()

    eb = s * EPT
    coff = c * N

    def gather(j, slot, sem):
        return pltpu.async_copy(hs.at[sv.at[pl.ds(j * K, K)]], rows.at[slot], sem)

    def gwait(slot, sem):
        pltpu.make_async_copy(hs.at[sv.at[pl.ds(0, K)]], rows.at[slot], sem).wait()

    def swait(slot, idx, sem):
        pltpu.make_async_copy(rows.at[slot], acc.at[idx], sem).wait()

    def phase(p, _):
        base = eb + p * IDS_PP
        pltpu.sync_copy(edges.at[pl.ds(base, IDS_PP)], sv)
        pltpu.sync_copy(edges.at[pl.ds(E + base, IDS_PP)], dv)

        def addoff(t, _):
            sv[pl.ds(t * 16, 16)] = sv[pl.ds(t * 16, 16)] + coff
            return 0
        lax.fori_loop(0, IDS_PP // 16, addoff, 0)

        _refill(didx0, dv, 0)
        _refill(didx1, dv, 1)
        gather(0, 0, gsem0)
        gather(1, 1, gsem1)

        def body(i, _):
            a = 2 * i
            gwait(0, gsem0)
            pltpu.async_copy(rows.at[0], acc.at[didx0], ssem0, add=True)
            gwait(1, gsem1)
            pltpu.async_copy(rows.at[1], acc.at[didx1], ssem1, add=True)
            swait(0, didx0, ssem0)

            @pl.when(i < CPP // 2 - 1)
            def _():
                _refill(didx0, dv, a + 2)
                gather(a + 2, 0, gsem0)
            swait(1, didx1, ssem1)

            @pl.when(i < CPP // 2 - 1)
            def _():
                _refill(didx1, dv, a + 3)
                gather(a + 3, 1, gsem1)
            return 0
        lax.fori_loop(0, CPP // 2, body, 0)
        return 0
    lax.fori_loop(0, PHASES, phase, 0)
    plsc.subcore_barrier()

    @pl.when(s < 15)
    def _():
        pltpu.sync_copy(acc.at[pl.ds(s * ROWS_PER_TILE, ROWS_PER_TILE)],
                        out.at[pl.ds(c * N + s * ROWS_PER_TILE, ROWS_PER_TILE)])

    @pl.when(s == 15)
    def _():
        pltpu.sync_copy(acc.at[pl.ds(15 * ROWS_PER_TILE, N - 15 * ROWS_PER_TILE)],
                        out.at[pl.ds(c * N + 15 * ROWS_PER_TILE, N - 15 * ROWS_PER_TILE)])


_sc_agg = functools.partial(
    pl.kernel,
    mesh=plsc.VectorSubcoreMesh(core_axis_name="c", subcore_axis_name="s"),
    out_type=jax.ShapeDtypeStruct((2 * N, HALF), jnp.float32),
    scratch_types=[
        pltpu.VMEM((IDS_PP,), jnp.int32),
        pltpu.VMEM((IDS_PP,), jnp.int32),
        pltpu.VMEM((K,), jnp.int32),
        pltpu.VMEM((K,), jnp.int32),
        pltpu.VMEM((2, K, HALF), jnp.float32),
        pltpu.VMEM_SHARED((NPAD, HALF), jnp.float32),
        pltpu.SemaphoreType.DMA,
        pltpu.SemaphoreType.DMA,
        pltpu.SemaphoreType.DMA,
        pltpu.SemaphoreType.DMA,
    ],
)(_sc_agg_body)


def _sc_row0_body(edges, hs, out, sv, dv, idxb, grows, accv, red, sgrid, gsem):
    c = lax.axis_index("c")
    s = lax.axis_index("s")
    zero16 = jnp.zeros((16,), jnp.float32)

    eb = s * EPT
    coff = c * N
    pltpu.sync_copy(edges.at[pl.ds(eb, EPT)], sv)
    pltpu.sync_copy(edges.at[pl.ds(E + eb, EPT)], dv)

    for col in range(HALF // 16):
        accv[0, pl.ds(col * 16, 16)] = zero16

    izero16 = jnp.zeros((16,), jnp.int32)

    def zmv(t, _):
        idxb[pl.ds(t * 16, 16)] = izero16
        return 0
    lax.fori_loop(0, EPT // 16 + 1, zmv, 0)

    def scan16(t, cnt):
        d16 = dv[pl.ds(t * 16, 16)]
        s16 = sv[pl.ds(t * 16, 16)] + coff
        for r in range(16):
            dr = d16[r]
            sr = s16[r]
            cur = cnt

            @pl.when(dr == 0)
            def _():
                idxb[pl.ds(cur, 16)] = jnp.full((16,), sr, jnp.int32)
            cnt = jnp.where(dr == 0, cnt + 1, cnt)
        return cnt
    nmatch = lax.fori_loop(0, EPT // 16, scan16, jnp.int32(0))

    def chunk(j, _):
        pltpu.async_copy(hs.at[idxb.at[pl.ds(j * K, K)]], grows, gsem).wait()

        def addrow(r, _):
            @pl.when(j * K + r < nmatch)
            def _():
                for col in range(HALF // 16):
                    sl = pl.ds(col * 16, 16)
                    accv[0, sl] = accv[0, sl] + grows[r, sl]
            return 0
        lax.fori_loop(0, K, addrow, 0)
        return 0
    lax.fori_loop(0, (nmatch + K - 1) // K, chunk, 0)

    pltpu.sync_copy(accv, sgrid.at[pl.ds(s, 1)])
    plsc.subcore_barrier()

    @pl.when(s == 0)
    def _():
        pltpu.sync_copy(sgrid, red)
        for col in range(HALF // 16):
            sl = pl.ds(col * 16, 16)
            v = red[0, sl]
            for r in range(1, 16):
                v = v + red[r, sl]
            accv[0, sl] = v
        pltpu.sync_copy(accv, out.at[pl.ds(c, 1)])


_sc_row0 = functools.partial(
    pl.kernel,
    mesh=plsc.VectorSubcoreMesh(core_axis_name="c", subcore_axis_name="s"),
    out_type=jax.ShapeDtypeStruct((2, HALF), jnp.float32),
    scratch_types=[
        pltpu.VMEM((EPT,), jnp.int32),
        pltpu.VMEM((EPT,), jnp.int32),
        pltpu.VMEM((EPT + 16,), jnp.int32),
        pltpu.VMEM((K, HALF), jnp.float32),
        pltpu.VMEM((1, HALF), jnp.float32),
        pltpu.VMEM((16, HALF), jnp.float32),
        pltpu.VMEM_SHARED((16, HALF), jnp.float32),
        pltpu.SemaphoreType.DMA,
    ],
)(_sc_row0_body)


BN = 1000


def _tc_entry_body(x_ref, wf_ref, bf_ref, dgo_ref, dgi_ref,
                   h_ref, hs_ref, ns_ref, nd_ref):
    ns = lax.rsqrt(jnp.maximum(dgo_ref[...], 1.0))
    nd = lax.rsqrt(jnp.maximum(dgi_ref[...], 1.0))
    h = jnp.dot(x_ref[...], wf_ref[...], preferred_element_type=jnp.float32)
    h = jnp.maximum(h + bf_ref[...], 0.0)
    hs = h * ns
    h_ref[0] = h[:, :HALF]
    h_ref[1] = h[:, HALF:]
    hs_ref[0] = hs[:, :HALF]
    hs_ref[1] = hs[:, HALF:]
    ns_ref[...] = ns
    nd_ref[...] = nd


_tc_entry = pl.pallas_call(
    _tc_entry_body,
    grid=(N // BN,),
    in_specs=[
        pl.BlockSpec((BN, IN_DIM), lambda i: (i, 0)),
        pl.BlockSpec((IN_DIM, LATENT), lambda i: (0, 0)),
        pl.BlockSpec((1, LATENT), lambda i: (0, 0)),
        pl.BlockSpec((BN, 1), lambda i: (i, 0)),
        pl.BlockSpec((BN, 1), lambda i: (i, 0)),
    ],
    out_specs=[
        pl.BlockSpec((2, BN, HALF), lambda i: (0, i, 0)),
        pl.BlockSpec((2, BN, HALF), lambda i: (0, i, 0)),
        pl.BlockSpec((BN, 1), lambda i: (i, 0)),
        pl.BlockSpec((BN, 1), lambda i: (i, 0)),
    ],
    out_shape=[
        jax.ShapeDtypeStruct((2, N, HALF), jnp.float32),
        jax.ShapeDtypeStruct((2, N, HALF), jnp.float32),
        jax.ShapeDtypeStruct((N, 1), jnp.float32),
        jax.ShapeDtypeStruct((N, 1), jnp.float32),
    ],
)


def _tc_layer_body(agg_ref, h_ref, nd_ref, ns_ref, w_ref, b_ref,
                   ho_ref, hso_ref):
    agg = jnp.concatenate([agg_ref[0], agg_ref[1]], axis=1) * nd_ref[...]
    y = jnp.dot(agg, w_ref[...], preferred_element_type=jnp.float32)
    y = jnp.maximum(y + b_ref[...], 0.0)
    h = jnp.concatenate([h_ref[0], h_ref[1]], axis=1) + y
    hs = h * ns_ref[...]
    ho_ref[0] = h[:, :HALF]
    ho_ref[1] = h[:, HALF:]
    hso_ref[0] = hs[:, :HALF]
    hso_ref[1] = hs[:, HALF:]


_tc_layer = pl.pallas_call(
    _tc_layer_body,
    grid=(N // BN,),
    in_specs=[
        pl.BlockSpec((2, BN, HALF), lambda i: (0, i, 0)),
        pl.BlockSpec((2, BN, HALF), lambda i: (0, i, 0)),
        pl.BlockSpec((BN, 1), lambda i: (i, 0)),
        pl.BlockSpec((BN, 1), lambda i: (i, 0)),
        pl.BlockSpec((LATENT, LATENT), lambda i: (0, 0)),
        pl.BlockSpec((1, LATENT), lambda i: (0, 0)),
    ],
    out_specs=[
        pl.BlockSpec((2, BN, HALF), lambda i: (0, i, 0)),
        pl.BlockSpec((2, BN, HALF), lambda i: (0, i, 0)),
    ],
    out_shape=[
        jax.ShapeDtypeStruct((2, N, HALF), jnp.float32),
        jax.ShapeDtypeStruct((2, N, HALF), jnp.float32),
    ],
)


def _tc_last_body(hrow_ref, aggr_ref, nd_ref, w_ref, b_ref, we_ref, be_ref,
                  o_ref):
    agg = aggr_ref[...] * nd_ref[0, 0]
    y = jnp.dot(agg, w_ref[...], preferred_element_type=jnp.float32)
    y = jnp.maximum(y + b_ref[...], 0.0)
    h0 = jnp.concatenate([hrow_ref[0, 0:1, :], hrow_ref[1, 0:1, :]], axis=1) + y
    o = jnp.dot(h0, we_ref[...], preferred_element_type=jnp.float32)
    o_ref[...] = jnp.maximum(o + be_ref[...], 0.0)


_tc_last = pl.pallas_call(
    _tc_last_body,
    out_shape=jax.ShapeDtypeStruct((1, EMB), jnp.float32),
)


def kernel(x, edge_index, W_f, b_f, W_g, b_g, W_h, b_h, W_e, b_e):
    edges_flat = edge_index.reshape(2 * E)
    degs = _sc_deg(edges_flat)
    dgo = degs[0:N, 0].reshape(N, 1)
    dgi = degs[N:2 * N, 0].reshape(N, 1)

    h, hs, ns, nd = _tc_entry(x, W_f, b_f.reshape(1, LATENT), dgo, dgi)
    bh = W_h, b_h.reshape(1, LATENT)
    for W, bb, steps in ((W_g, b_g.reshape(1, LATENT), 2), (W_h, bh[1], 3)):
        for _ in range(steps):
            agg = _sc_agg(edges_flat, hs.reshape(2 * N, HALF))
            h, hs = _tc_layer(agg.reshape(2, N, HALF), h, nd, ns, W, bb)

    aggr = _sc_row0(edges_flat, hs.reshape(2 * N, HALF))
    out1 = _tc_last(h[:, 0:8, :], aggr.reshape(1, LATENT), nd[0:8],
                    W_h, bh[1], W_e, b_e.reshape(1, EMB))
    return out1[0]


# speedup vs baseline: 6.6909x; 1.0065x over previous
import functools

import jax
import jax.numpy as jnp
from jax import lax
from jax.experimental import pallas as pl
from jax.experimental.pallas import tpu as pltpu
from jax.experimental.pallas import tpu_sc as plsc

N = 10000
E = 320000
IN_DIM = 128
LATENT = 256
EMB = 128
HALF = 128

NPAD = 10240
ROWS_PER_TILE = NPAD // 16
EPT = E // 16
K = 80
CHUNKS = EPT // K
PHASES = 5
CPP = CHUNKS // PHASES
IDS_PP = CPP * K


def _refill(idx, ev, j):
    def go(t, _):
        idx[pl.ds(t * 16, 16)] = ev[pl.ds(j * K + t * 16, 16)]
        return 0
    lax.fori_loop(0, K // 16, go, 0)


def _sc_deg_body(edges, out, ev, idx0, idx1, ones, acc, ssem0, ssem1):
    c = lax.axis_index("c")
    s = lax.axis_index("s")
    zero16 = jnp.zeros((16,), jnp.float32)

    def zrow(t, _):
        ones[t // 8, pl.ds((t % 8) * 16, 16)] = zero16
        return 0
    lax.fori_loop(0, K * 8, zrow, 0)

    def zacc(i, _):
        pltpu.sync_copy(ones, acc.at[pl.ds(s * ROWS_PER_TILE + i * K, K)])
        return 0
    lax.fori_loop(0, ROWS_PER_TILE // K, zacc, 0)
    plsc.subcore_barrier()

    onehot = jnp.where(lax.iota(jnp.int32, 16) == 0, 1.0, 0.0).astype(jnp.float32)

    def srow(i, _):
        ones[i, pl.ds(0, 16)] = onehot
        return 0
    lax.fori_loop(0, K, srow, 0)

    pltpu.sync_copy(edges.at[pl.ds(c * E + s * EPT, EPT)], ev)
    _refill(idx0, ev, 0)

    def body(i, _):
        a = 2 * i
        pltpu.async_copy(ones, acc.at[idx0], ssem0, add=True)

        @pl.when(i > 0)
        def _():
            pltpu.make_async_copy(ones, acc.at[idx1], ssem1).wait()
        _refill(idx1, ev, a + 1)
        pltpu.async_copy(ones, acc.at[idx1], ssem1, add=True)
        pltpu.make_async_copy(ones, acc.at[idx0], ssem0).wait()

        @pl.when(i < CHUNKS // 2 - 1)
        def _():
            _refill(idx0, ev, a + 2)
        return 0
    lax.fori_loop(0, CHUNKS // 2, body, 0)
    pltpu.make_async_copy(ones, acc.at[idx1], ssem1).wait()
    plsc.subcore_barrier()

    @pl.when(s < 15)
    def _():
        pltpu.sync_copy(acc.at[pl.ds(s * ROWS_PER_TILE, ROWS_PER_TILE)],
                        out.at[pl.ds(c * N + s * ROWS_PER_TILE, ROWS_PER_TILE)])

    @pl.when(s == 15)
    def _():
        pltpu.sync_copy(acc.at[pl.ds(15 * ROWS_PER_TILE, N - 15 * ROWS_PER_TILE)],
                        out.at[pl.ds(c * N + 15 * ROWS_PER_TILE, N - 15 * ROWS_PER_TILE)])


_sc_deg = functools.partial(
    pl.kernel,
    mesh=plsc.VectorSubcoreMesh(core_axis_name="c", subcore_axis_name="s"),
    out_type=jax.ShapeDtypeStruct((2 * N, HALF), jnp.float32),
    scratch_types=[
        pltpu.VMEM((EPT,), jnp.int32),
        pltpu.VMEM((K,), jnp.int32),
        pltpu.VMEM((K,), jnp.int32),
        pltpu.VMEM((K, HALF), jnp.float32),
        pltpu.VMEM_SHARED((NPAD, HALF), jnp.float32),
        pltpu.SemaphoreType.DMA,
        pltpu.SemaphoreType.DMA,
    ],
)(_sc_deg_body)


def _sc_agg_body(edges, hs, out, sv, dv, didx0, didx1, rows, acc,
                 gsem0, gsem1, ssem0, ssem1):
    c = lax.axis_index("c")
    s = lax.axis_index("s")
    zero16 = jnp.zeros((16,), jnp.float32)

    def zrow(t, _):
        rows[0, t // 8, pl.ds((t % 8) * 16, 16)] = zero16
        return 0
    lax.fori_loop(0, K * 8, zrow, 0)

    def zacc(i, _):
        pltpu.sync_copy(rows.at[0], acc.at[pl.ds(s * ROWS_PER_TILE + i * K, K)])
        return 0
    lax.fori_loop(0, ROWS_PER_TILE // K, zacc, 0)
    plsc.subcore_barrier()

    eb = s * EPT
    coff = c * N

    def gather(j, slot, sem):
        return pltpu.async_copy(hs.at[sv.at[pl.ds(j * K, K)]], rows.at[slot], sem)

    def gwait(slot, sem):
        pltpu.make_async_copy(hs.at[sv.at[pl.ds(0, K)]], rows.at[slot], sem).wait()

    def swait(slot, idx, sem):
        pltpu.make_async_copy(rows.at[slot], acc.at[idx], sem).wait()

    def phase(p, _):
        base = eb + p * IDS_PP
        pltpu.sync_copy(edges.at[pl.ds(base, IDS_PP)], sv)
        pltpu.sync_copy(edges.at[pl.ds(E + base, IDS_PP)], dv)

        def addoff(t, _):
            sv[pl.ds(t * 16, 16)] = sv[pl.ds(t * 16, 16)] + coff
            return 0
        lax.fori_loop(0, IDS_PP // 16, addoff, 0)

        _refill(didx0, dv, 0)
        _refill(didx1, dv, 1)
        gather(0, 0, gsem0)
        gather(1, 1, gsem1)

        def body(i, _):
            a = 2 * i
            gwait(0, gsem0)
            pltpu.async_copy(rows.at[0], acc.at[didx0], ssem0, add=True)
            gwait(1, gsem1)
            pltpu.async_copy(rows.at[1], acc.at[didx1], ssem1, add=True)
            swait(0, didx0, ssem0)

            @pl.when(i < CPP // 2 - 1)
            def _():
                _refill(didx0, dv, a + 2)
                gather(a + 2, 0, gsem0)
            swait(1, didx1, ssem1)

            @pl.when(i < CPP // 2 - 1)
            def _():
                _refill(didx1, dv, a + 3)
                gather(a + 3, 1, gsem1)
            return 0
        lax.fori_loop(0, CPP // 2, body, 0)
        return 0
    lax.fori_loop(0, PHASES, phase, 0)
    plsc.subcore_barrier()

    @pl.when(s < 15)
    def _():
        pltpu.sync_copy(acc.at[pl.ds(s * ROWS_PER_TILE, ROWS_PER_TILE)],
                        out.at[pl.ds(c * N + s * ROWS_PER_TILE, ROWS_PER_TILE)])

    @pl.when(s == 15)
    def _():
        pltpu.sync_copy(acc.at[pl.ds(15 * ROWS_PER_TILE, N - 15 * ROWS_PER_TILE)],
                        out.at[pl.ds(c * N + 15 * ROWS_PER_TILE, N - 15 * ROWS_PER_TILE)])


_sc_agg = functools.partial(
    pl.kernel,
    mesh=plsc.VectorSubcoreMesh(core_axis_name="c", subcore_axis_name="s"),
    out_type=jax.ShapeDtypeStruct((2 * N, HALF), jnp.float32),
    scratch_types=[
        pltpu.VMEM((IDS_PP,), jnp.int32),
        pltpu.VMEM((IDS_PP,), jnp.int32),
        pltpu.VMEM((K,), jnp.int32),
        pltpu.VMEM((K,), jnp.int32),
        pltpu.VMEM((2, K, HALF), jnp.float32),
        pltpu.VMEM_SHARED((NPAD, HALF), jnp.float32),
        pltpu.SemaphoreType.DMA,
        pltpu.SemaphoreType.DMA,
        pltpu.SemaphoreType.DMA,
        pltpu.SemaphoreType.DMA,
    ],
)(_sc_agg_body)


def _sc_row0_body(edges, hs, out, sv, dv, idxb, grows, accv, red, sgrid, gsem):
    c = lax.axis_index("c")
    s = lax.axis_index("s")
    zero16 = jnp.zeros((16,), jnp.float32)

    eb = s * EPT
    coff = c * N
    pltpu.sync_copy(edges.at[pl.ds(eb, EPT)], sv)
    pltpu.sync_copy(edges.at[pl.ds(E + eb, EPT)], dv)

    for col in range(HALF // 16):
        accv[0, pl.ds(col * 16, 16)] = zero16

    izero16 = jnp.zeros((16,), jnp.int32)

    def zmv(t, _):
        idxb[pl.ds(t * 16, 16)] = izero16
        return 0
    lax.fori_loop(0, EPT // 16 + 1, zmv, 0)

    def scan16(t, cnt):
        d16 = dv[pl.ds(t * 16, 16)]
        s16 = sv[pl.ds(t * 16, 16)] + coff
        for r in range(16):
            dr = d16[r]
            sr = s16[r]
            cur = cnt

            @pl.when(dr == 0)
            def _():
                idxb[pl.ds(cur, 16)] = jnp.full((16,), sr, jnp.int32)
            cnt = jnp.where(dr == 0, cnt + 1, cnt)
        return cnt
    nmatch = lax.fori_loop(0, EPT // 16, scan16, jnp.int32(0))

    def chunk(j, _):
        pltpu.async_copy(hs.at[idxb.at[pl.ds(j * K, K)]], grows, gsem).wait()

        def addrow(r, _):
            @pl.when(j * K + r < nmatch)
            def _():
                for col in range(HALF // 16):
                    sl = pl.ds(col * 16, 16)
                    accv[0, sl] = accv[0, sl] + grows[r, sl]
            return 0
        lax.fori_loop(0, K, addrow, 0)
        return 0
    lax.fori_loop(0, (nmatch + K - 1) // K, chunk, 0)

    pltpu.sync_copy(accv, sgrid.at[pl.ds(s, 1)])
    plsc.subcore_barrier()

    @pl.when(s == 0)
    def _():
        pltpu.sync_copy(sgrid, red)
        for col in range(HALF // 16):
            sl = pl.ds(col * 16, 16)
            v = red[0, sl]
            for r in range(1, 16):
                v = v + red[r, sl]
            accv[0, sl] = v
        pltpu.sync_copy(accv, out.at[pl.ds(c, 1)])


_sc_row0 = functools.partial(
    pl.kernel,
    mesh=plsc.VectorSubcoreMesh(core_axis_name="c", subcore_axis_name="s"),
    out_type=jax.ShapeDtypeStruct((2, HALF), jnp.float32),
    scratch_types=[
        pltpu.VMEM((EPT,), jnp.int32),
        pltpu.VMEM((EPT,), jnp.int32),
        pltpu.VMEM((EPT + 16,), jnp.int32),
        pltpu.VMEM((K, HALF), jnp.float32),
        pltpu.VMEM((1, HALF), jnp.float32),
        pltpu.VMEM((16, HALF), jnp.float32),
        pltpu.VMEM_SHARED((16, HALF), jnp.float32),
        pltpu.SemaphoreType.DMA,
    ],
)(_sc_row0_body)


BN = 2000


def _tc_entry_body(x_ref, wf_ref, bf_ref, dgo_ref, dgi_ref,
                   h_ref, hs_ref, ns_ref, nd_ref):
    ns = lax.rsqrt(jnp.maximum(dgo_ref[...], 1.0))
    nd = lax.rsqrt(jnp.maximum(dgi_ref[...], 1.0))
    h = jnp.dot(x_ref[...], wf_ref[...], preferred_element_type=jnp.float32)
    h = jnp.maximum(h + bf_ref[...], 0.0)
    hs = h * ns
    h_ref[0] = h[:, :HALF]
    h_ref[1] = h[:, HALF:]
    hs_ref[0] = hs[:, :HALF]
    hs_ref[1] = hs[:, HALF:]
    ns_ref[...] = ns
    nd_ref[...] = nd


_tc_entry = pl.pallas_call(
    _tc_entry_body,
    grid=(N // BN,),
    in_specs=[
        pl.BlockSpec((BN, IN_DIM), lambda i: (i, 0)),
        pl.BlockSpec((IN_DIM, LATENT), lambda i: (0, 0)),
        pl.BlockSpec((1, LATENT), lambda i: (0, 0)),
        pl.BlockSpec((BN, 1), lambda i: (i, 0)),
        pl.BlockSpec((BN, 1), lambda i: (i, 0)),
    ],
    out_specs=[
        pl.BlockSpec((2, BN, HALF), lambda i: (0, i, 0)),
        pl.BlockSpec((2, BN, HALF), lambda i: (0, i, 0)),
        pl.BlockSpec((BN, 1), lambda i: (i, 0)),
        pl.BlockSpec((BN, 1), lambda i: (i, 0)),
    ],
    out_shape=[
        jax.ShapeDtypeStruct((2, N, HALF), jnp.float32),
        jax.ShapeDtypeStruct((2, N, HALF), jnp.float32),
        jax.ShapeDtypeStruct((N, 1), jnp.float32),
        jax.ShapeDtypeStruct((N, 1), jnp.float32),
    ],
)


def _tc_layer_body(agg_ref, h_ref, nd_ref, ns_ref, w_ref, b_ref,
                   ho_ref, hso_ref):
    agg = jnp.concatenate([agg_ref[0], agg_ref[1]], axis=1) * nd_ref[...]
    y = jnp.dot(agg, w_ref[...], preferred_element_type=jnp.float32)
    y = jnp.maximum(y + b_ref[...], 0.0)
    h = jnp.concatenate([h_ref[0], h_ref[1]], axis=1) + y
    hs = h * ns_ref[...]
    ho_ref[0] = h[:, :HALF]
    ho_ref[1] = h[:, HALF:]
    hso_ref[0] = hs[:, :HALF]
    hso_ref[1] = hs[:, HALF:]


_tc_layer = pl.pallas_call(
    _tc_layer_body,
    grid=(N // BN,),
    in_specs=[
        pl.BlockSpec((2, BN, HALF), lambda i: (0, i, 0)),
        pl.BlockSpec((2, BN, HALF), lambda i: (0, i, 0)),
        pl.BlockSpec((BN, 1), lambda i: (i, 0)),
        pl.BlockSpec((BN, 1), lambda i: (i, 0)),
        pl.BlockSpec((LATENT, LATENT), lambda i: (0, 0)),
        pl.BlockSpec((1, LATENT), lambda i: (0, 0)),
    ],
    out_specs=[
        pl.BlockSpec((2, BN, HALF), lambda i: (0, i, 0)),
        pl.BlockSpec((2, BN, HALF), lambda i: (0, i, 0)),
    ],
    out_shape=[
        jax.ShapeDtypeStruct((2, N, HALF), jnp.float32),
        jax.ShapeDtypeStruct((2, N, HALF), jnp.float32),
    ],
)


def _tc_last_body(hrow_ref, aggr_ref, nd_ref, w_ref, b_ref, we_ref, be_ref,
                  o_ref):
    agg = aggr_ref[...] * nd_ref[0, 0]
    y = jnp.dot(agg, w_ref[...], preferred_element_type=jnp.float32)
    y = jnp.maximum(y + b_ref[...], 0.0)
    h0 = jnp.concatenate([hrow_ref[0, 0:1, :], hrow_ref[1, 0:1, :]], axis=1) + y
    o = jnp.dot(h0, we_ref[...], preferred_element_type=jnp.float32)
    o_ref[...] = jnp.maximum(o + be_ref[...], 0.0)


_tc_last = pl.pallas_call(
    _tc_last_body,
    out_shape=jax.ShapeDtypeStruct((1, EMB), jnp.float32),
)


def kernel(x, edge_index, W_f, b_f, W_g, b_g, W_h, b_h, W_e, b_e):
    edges_flat = edge_index.reshape(2 * E)
    degs = _sc_deg(edges_flat)
    dgo = degs[0:N, 0].reshape(N, 1)
    dgi = degs[N:2 * N, 0].reshape(N, 1)

    h, hs, ns, nd = _tc_entry(x, W_f, b_f.reshape(1, LATENT), dgo, dgi)
    bh = W_h, b_h.reshape(1, LATENT)
    for W, bb, steps in ((W_g, b_g.reshape(1, LATENT), 2), (W_h, bh[1], 3)):
        for _ in range(steps):
            agg = _sc_agg(edges_flat, hs.reshape(2 * N, HALF))
            h, hs = _tc_layer(agg.reshape(2, N, HALF), h, nd, ns, W, bb)

    aggr = _sc_row0(edges_flat, hs.reshape(2 * N, HALF))
    out1 = _tc_last(h[:, 0:8, :], aggr.reshape(1, LATENT), nd[0:8],
                    W_h, bh[1], W_e, b_e.reshape(1, EMB))
    return out1[0]
